# baseline (speedup 1.0000x reference)
import functools

import jax
import jax.numpy as jnp
from jax import lax
from jax.experimental import pallas as pl
from jax.experimental.pallas import tpu as pltpu
from jax.experimental.pallas import tpu_sc as plsc

_N = 65536
_NW = 32
_W = _N // _NW
_C = 512
_SEGS = 4


def _sc_gather(feats_hbm, hand_hbm, suit_hbm, bid_hbm, role_hbm, emb_hbm,
               feats_v, emb_v, hand_v, suit_v, bid_v, role_v):
    wid = lax.axis_index("s") * 2 + lax.axis_index("c")
    base = wid * _W

    pltpu.sync_copy(hand_hbm, hand_v)
    pltpu.sync_copy(suit_hbm, suit_v)
    pltpu.sync_copy(bid_hbm, bid_v)
    pltpu.sync_copy(role_hbm, role_v)

    lanes = lax.iota(jnp.int32, 16)

    def do_chunk(ch, _):
        cbase = base + ch * _C
        pltpu.sync_copy(feats_hbm.at[pl.ds(cbase * 15, _C * 15)],
                        feats_v.at[pl.ds(0, _C * 15)])

        def do_item(k, _):
            v = feats_v[pl.ds(k * 15, 16)]

            def row(tab_v, col):
                i = v[col].astype(jnp.int32)
                return plsc.load_gather(tab_v, [lanes + i * 16])

            hand = row(hand_v, 0)
            suit = (row(suit_v, 1) + row(suit_v, 2)
                    + row(suit_v, 3) + row(suit_v, 4)) * 0.25
            bid = row(bid_v, 5)
            role = row(role_v, 6)
            emb_v[pl.ds(k * 64, 16)] = hand
            emb_v[pl.ds(k * 64 + 16, 16)] = suit
            emb_v[pl.ds(k * 64 + 32, 16)] = bid
            emb_v[pl.ds(k * 64 + 48, 16)] = role
            return ()

        lax.fori_loop(0, _C, do_item, ())
        pltpu.sync_copy(emb_v, emb_hbm.at[pl.ds(cbase * 64, _C * 64)])
        return ()

    lax.fori_loop(0, _W // _C, do_chunk, ())


def sc_gather_embs(feats, hand_tab, suit_tab, bid_tab, role_tab):
    mesh = plsc.VectorSubcoreMesh(core_axis_name="c", subcore_axis_name="s")
    k = functools.partial(
        pl.kernel, mesh=mesh,
        compiler_params=pltpu.CompilerParams(needs_layout_passes=False),
        out_type=jax.ShapeDtypeStruct((_N * 64,), jnp.float32),
        scratch_types=[
            pltpu.VMEM((_C * 15 + 16,), jnp.float32),
            pltpu.VMEM((_C * 64,), jnp.float32),
            pltpu.VMEM((9 * 16,), jnp.float32),
            pltpu.VMEM((9 * 16,), jnp.float32),
            pltpu.VMEM((29 * 16,), jnp.float32),
            pltpu.VMEM((4 * 16,), jnp.float32),
        ],
    )(_sc_gather)
    return k(feats.reshape(_N * 15), hand_tab.reshape(9 * 16),
             suit_tab.reshape(9 * 16), bid_tab.reshape(29 * 16),
             role_tab.reshape(4 * 16)).reshape(_N, 64)


def _tc_kernel(feats_ref, emb_ref, wb1_ref, bb1_ref, wb2_ref, bb2_ref,
               wf1_ref, bf1_ref, wf2_ref, bf2_ref, out_ref, wbeh_s, bz_s):
    @pl.when(pl.program_id(0) == 0)
    def _fold():
        wf1_beh = wf1_ref[...][:, 64:128]
        wbeh_s[...] = lax.dot_general(
            wb2_ref[...], wf1_beh, (((0,), (1,)), ((), ())),
            preferred_element_type=jnp.float32)
        bz_s[...] = bf1_ref[...] + lax.dot_general(
            bb2_ref[...], wf1_beh, (((1,), (1,)), ((), ())),
            preferred_element_type=jnp.float32)

    f = feats_ref[...]
    h = jnp.maximum(
        lax.dot_general(f[:, 7:15], wb1_ref[...], (((1,), (1,)), ((), ())),
                        preferred_element_type=jnp.float32) + bb1_ref[...],
        0.0)
    z = (lax.dot_general(emb_ref[...], wf1_ref[...][:, 0:64],
                         (((1,), (1,)), ((), ())),
                         preferred_element_type=jnp.float32)
         + jnp.dot(h, wbeh_s[...], preferred_element_type=jnp.float32)
         + bz_s[...])
    g = jnp.maximum(z, 0.0)
    out_ref[...] = lax.dot_general(
        g, wf2_ref[...], (((1,), (1,)), ((), ())),
        preferred_element_type=jnp.float32) + bf2_ref[...]


@jax.jit
def kernel(player_features, hand_tab, suit_tab, bid_tab, role_tab,
           Wb1, bb1, Wb2, bb2, Wf1, bf1, Wf2, bf2):
    B, P, D = player_features.shape[0], player_features.shape[1], Wf1.shape[0]
    N = B * P
    feats = player_features.reshape(N, 15)
    emb = sc_gather_embs(feats, hand_tab, suit_tab, bid_tab, role_tab)

    R = 16384
    grid = (N // R,)

    def full(shape):
        return pl.BlockSpec(shape, lambda i: (0,) * len(shape))

    out = pl.pallas_call(
        _tc_kernel,
        grid=grid,
        in_specs=[
            pl.BlockSpec((R, 15), lambda i: (i, 0)),
            pl.BlockSpec((R, 64), lambda i: (i, 0)),
            full((64, 8)), full((1, 64)), full((64, 64)), full((1, 64)),
            full((128, 128)), full((1, 128)), full((128, 128)),
            full((1, 128)),
        ],
        out_specs=pl.BlockSpec((R, 128), lambda i: (i, 0)),
        out_shape=jax.ShapeDtypeStruct((N, D), jnp.float32),
        scratch_shapes=[
            pltpu.VMEM((64, 128), jnp.float32),
            pltpu.VMEM((1, 128), jnp.float32),
        ],
    )(feats, emb, Wb1, bb1.reshape(1, 64), Wb2, bb2.reshape(1, 64),
      Wf1, bf1.reshape(1, 128), Wf2, bf2.reshape(1, 128))
    return out.reshape(B, P, D)
